# baseline (speedup 1.0000x reference)
import functools

import jax
import jax.numpy as jnp
from jax import lax
from jax.experimental import pallas as pl
from jax.experimental.pallas import tpu as pltpu
from jax.experimental.pallas import tpu_sc as plsc

EMB = 64
LANES = 16
EMB_V = EMB // LANES
SUB = 100
CB = 2


def _make_pool(B, L, V):
    NC, NS = 2, 16
    NW = NC * NS
    b_per_w = B // NW
    nsub = CB * L // SUB
    nchunk = b_per_w // CB
    chunk_rows = CB * L
    idx_rows_per_w = b_per_w * L // SUB

    mesh = plsc.VectorSubcoreMesh(
        core_axis_name="c", subcore_axis_name="s", num_cores=NC, num_subcores=NS
    )

    @functools.partial(
        pl.kernel,
        out_type=jax.ShapeDtypeStruct((B, EMB), jnp.float32),
        mesh=mesh,
        scratch_types=[
            pltpu.VMEM((idx_rows_per_w, SUB), jnp.int32),
            pltpu.VMEM((2 * chunk_rows, EMB), jnp.float32),
            pltpu.VMEM((b_per_w, EMB), jnp.float32),
            pltpu.SemaphoreType.DMA,
        ],
        compiler_params=pltpu.CompilerParams(use_tc_tiling_on_sc=False),
    )
    def pool(x_hbm, table_hbm, out_hbm, idx_v, rows_v, out_v, sem):
        wid = lax.axis_index("s") * NC + lax.axis_index("c")
        pltpu.sync_copy(x_hbm.at[pl.ds(wid * idx_rows_per_w, idx_rows_per_w)], idx_v)

        def gathers(ck, buf):
            return [
                pltpu.make_async_copy(
                    table_hbm.at[idx_v.at[ck * nsub + i]],
                    rows_v.at[pl.ds(buf * chunk_rows + i * SUB, SUB)],
                    sem,
                )
                for i in range(nsub)
            ]

        def fire(ck, buf):
            for cp in gathers(ck, buf):
                cp.start()

        def drain(ck, buf):
            for cp in gathers(ck, buf):
                cp.wait()

        def reduce(ck, buf):
            for r in range(CB):
                base = buf * chunk_rows + r * L

                def red(j, acc, base=base):
                    row = base + j
                    return tuple(
                        acc[k] + rows_v[row, pl.ds(LANES * k, LANES)]
                        for k in range(EMB_V)
                    )

                acc = tuple(
                    rows_v[base, pl.ds(LANES * k, LANES)] for k in range(EMB_V)
                )
                acc = lax.fori_loop(1, L, red, acc, unroll=8)
                for k in range(EMB_V):
                    out_v[ck * CB + r, pl.ds(LANES * k, LANES)] = acc[k]

        fire(0, 0)

        def body(ck, carry):
            fire(ck + 1, (ck + 1) % 2)
            drain(ck, ck % 2)
            reduce(ck, ck % 2)
            return carry

        lax.fori_loop(0, nchunk - 1, body, 0)
        last = nchunk - 1
        drain(last, last % 2)
        reduce(last, last % 2)
        pltpu.sync_copy(out_v, out_hbm.at[pl.ds(wid * b_per_w, b_per_w)])

    return pool


def _mlp_body(inv_l, s_ref, w1_ref, b1_ref, w2_ref, b2_ref, o_ref):
    p = s_ref[...] * inv_l
    h = jnp.dot(p, w1_ref[...], preferred_element_type=jnp.float32) + b1_ref[...]
    o_ref[...] = (
        jnp.dot(h, w2_ref[...], preferred_element_type=jnp.float32) + b2_ref[...]
    )


def kernel(x, table, W1, b1, W2, b2):
    B, L = x.shape
    V, _ = table.shape
    x2d = x.reshape(B * L // SUB, SUB).astype(jnp.int32)
    sums = _make_pool(B, L, V)(x2d, table)
    out = pl.pallas_call(
        functools.partial(_mlp_body, 1.0 / L),
        out_shape=jax.ShapeDtypeStruct((B, W2.shape[1]), jnp.float32),
    )(sums, W1, b1.reshape(1, -1), W2, b2.reshape(1, -1))
    return out


# speedup vs baseline: 1.0053x; 1.0053x over previous
import functools

import jax
import jax.numpy as jnp
from jax import lax
from jax.experimental import pallas as pl
from jax.experimental.pallas import tpu as pltpu
from jax.experimental.pallas import tpu_sc as plsc

EMB = 64
LANES = 16
EMB_V = EMB // LANES
SUB = 40
CB = 2


def _make_pool(B, L, V):
    NC, NS = 2, 16
    NW = NC * NS
    b_per_w = B // NW
    nsub = CB * L // SUB
    nchunk = b_per_w // CB
    chunk_rows = CB * L

    mesh = plsc.VectorSubcoreMesh(
        core_axis_name="c", subcore_axis_name="s", num_cores=NC, num_subcores=NS
    )

    @functools.partial(
        pl.kernel,
        out_type=jax.ShapeDtypeStruct((B, EMB), jnp.float32),
        mesh=mesh,
        scratch_types=[
            pltpu.VMEM((b_per_w, L), jnp.int32),
            pltpu.VMEM((2 * chunk_rows, EMB), jnp.float32),
            pltpu.VMEM((b_per_w, EMB), jnp.float32),
            pltpu.SemaphoreType.DMA,
        ],
        compiler_params=pltpu.CompilerParams(use_tc_tiling_on_sc=False),
    )
    def pool(x_hbm, table_hbm, out_hbm, idx_v, rows_v, out_v, sem):
        wid = lax.axis_index("s") * NC + lax.axis_index("c")
        pltpu.sync_copy(x_hbm.at[pl.ds(wid * b_per_w, b_per_w)], idx_v)

        def gathers(ck, buf):
            return [
                pltpu.make_async_copy(
                    table_hbm.at[idx_v.at[ck * CB + i // (L // SUB),
                                          pl.ds((i % (L // SUB)) * SUB, SUB)]],
                    rows_v.at[pl.ds(buf * chunk_rows + i * SUB, SUB)],
                    sem,
                )
                for i in range(nsub)
            ]

        def fire(ck, buf):
            for cp in gathers(ck, buf):
                cp.start()

        def drain(ck, buf):
            for cp in gathers(ck, buf):
                cp.wait()

        def reduce(ck, buf):
            for r in range(CB):
                base = buf * chunk_rows + r * L

                def red(j, acc, base=base):
                    row = base + j
                    return tuple(
                        acc[k] + rows_v[row, pl.ds(LANES * k, LANES)]
                        for k in range(EMB_V)
                    )

                acc = tuple(
                    rows_v[base, pl.ds(LANES * k, LANES)] for k in range(EMB_V)
                )
                acc = lax.fori_loop(1, L, red, acc, unroll=8)
                for k in range(EMB_V):
                    out_v[ck * CB + r, pl.ds(LANES * k, LANES)] = acc[k]

        fire(0, 0)

        def body(ck, carry):
            fire(ck + 1, (ck + 1) % 2)
            drain(ck, ck % 2)
            reduce(ck, ck % 2)
            return carry

        lax.fori_loop(0, nchunk - 1, body, 0)
        last = nchunk - 1
        drain(last, last % 2)
        reduce(last, last % 2)
        pltpu.sync_copy(out_v, out_hbm.at[pl.ds(wid * b_per_w, b_per_w)])

    return pool


def _mlp_body(inv_l, s_ref, w1_ref, b1_ref, w2_ref, b2_ref, o_ref):
    p = s_ref[...] * inv_l
    h = jnp.dot(p, w1_ref[...], preferred_element_type=jnp.float32) + b1_ref[...]
    o_ref[...] = (
        jnp.dot(h, w2_ref[...], preferred_element_type=jnp.float32) + b2_ref[...]
    )


def kernel(x, table, W1, b1, W2, b2):
    B, L = x.shape
    V, _ = table.shape
    sums = _make_pool(B, L, V)(x.astype(jnp.int32), table)
    out = pl.pallas_call(
        functools.partial(_mlp_body, 1.0 / L),
        out_shape=jax.ShapeDtypeStruct((B, W2.shape[1]), jnp.float32),
    )(sums, W1, b1.reshape(1, -1), W2, b2.reshape(1, -1))
    return out
